# baseline (speedup 1.0000x reference)
import jax
import jax.numpy as jnp
from jax.experimental import pallas as pl
from jax.experimental.pallas import tpu as pltpu

_NUM_HEADS = 12
_HEAD_DIM = 64
_EMBED = 768
_GROUP = 4
_CHUNK = 2048
_HPP = 2
_W = _HPP * _HEAD_DIM
_INV_SQRT = 1.0 / (_HEAD_DIM ** 0.5)
_SCALE = _INV_SQRT * 1.4426950408889634

_DN_T = (((1,), (1,)), ((), ()))
_DN_N = (((1,), (0,)), ((), ()))


def _dequant_keys(k):
    a = jnp.abs(k)
    n = k.shape[1]
    w = a
    for r in (1, 2, 3):
        w = jnp.maximum(w, pltpu.roll(a, n - r, 1))
    lane = jax.lax.broadcasted_iota(jnp.int32, k.shape, 1)
    first = (lane % _GROUP) == 0
    m0 = jnp.where(first, w, 0.0)
    gm = m0
    for r in (1, 2, 3):
        gm = gm + pltpu.roll(m0, r, 1)
    scale = jnp.where(gm > 0.0, gm, 1.0)
    kq = jnp.clip(jnp.round(k / scale), -2.0, 1.0)
    return kq * scale


def _attn_kernel(x_ref, wq_ref, wk_ref, wv_ref, bq_ref, bk_ref, bv_ref,
                 wp_ref, bp_ref, out_ref):
    g = pl.program_id(0)

    @pl.when(g == 0)
    def _init():
        out_ref[...] = jnp.broadcast_to(bp_ref[...], out_ref.shape)

    x = x_ref[...]
    wq, wk, wv = wq_ref[...], wk_ref[...], wv_ref[...]
    bq, bk, bv = bq_ref[0], bk_ref[0], bv_ref[0]
    xb = x.astype(jnp.bfloat16)
    k2 = jax.lax.dot_general(x, wk, _DN_N) + bk
    v2 = jax.lax.dot_general(xb, wv.astype(jnp.bfloat16), _DN_N,
                             preferred_element_type=jnp.float32) + bv
    kd2 = _dequant_keys(k2)
    kd2b = kd2.astype(jnp.bfloat16)
    wqb = wq.astype(jnp.bfloat16)
    wpb = wp_ref[...].astype(jnp.bfloat16)
    lane = jax.lax.broadcasted_iota(jnp.int32, v2.shape, 1)
    vaug = [
        jnp.where(lane < _HEAD_DIM, v2,
                  jnp.where(lane == _HEAD_DIM, 1.0, 0.0)).astype(jnp.bfloat16),
        jnp.where(lane >= _HEAD_DIM, v2,
                  jnp.where(lane == 0, 1.0, 0.0)).astype(jnp.bfloat16),
    ]
    cm = jnp.max(jnp.abs(kd2), axis=0, keepdims=True)
    cmt = jnp.broadcast_to(cm, (_W, _W)).T
    row_i = jax.lax.broadcasted_iota(jnp.int32, (_W, _W), 0)
    col_i = jax.lax.broadcasted_iota(jnp.int32, (_W, _W), 1)
    sel = ((col_i == 0) & (row_i < _HEAD_DIM)) | \
          ((col_i == _HEAD_DIM) & (row_i >= _HEAD_DIM))
    cmm = jnp.where(sel, cmt, 0.0)
    seq = x.shape[0]
    for c in range(seq // _CHUNK):
        rows = pl.ds(c * _CHUNK, _CHUNK)
        xcb = xb[c * _CHUNK:(c + 1) * _CHUNK, :]
        q2 = (jax.lax.dot_general(xcb, wqb, _DN_N,
                                  preferred_element_type=jnp.float32)
              + bq) * _SCALE
        q2b = q2.astype(jnp.bfloat16)
        mm = jax.lax.dot_general(jnp.abs(q2), cmm, _DN_N)
        outs = []
        for hh in range(_HPP):
            lanes = slice(hh * _HEAD_DIM, (hh + 1) * _HEAD_DIM)
            s = jax.lax.dot_general(q2b[:, lanes], kd2b[:, lanes], _DN_T,
                                    preferred_element_type=jnp.float32)
            m = mm[:, 0:1] if hh == 0 else mm[:, _HEAD_DIM:_HEAD_DIM + 1]
            e = jnp.exp2(s - m)
            eb = e.astype(jnp.bfloat16)
            o_raw = jax.lax.dot_general(
                eb, vaug[hh], (((1,), (0,)), ((), ())),
                preferred_element_type=jnp.float32)
            den = o_raw[:, _HEAD_DIM:_HEAD_DIM + 1] if hh == 0 else o_raw[:, 0:1]
            outs.append(o_raw[:, lanes] / den)
        o2 = jnp.concatenate(outs, axis=1).astype(jnp.bfloat16)
        out_ref[rows, :] += jax.lax.dot_general(
            o2, wpb, (((1,), (0,)), ((), ())),
            preferred_element_type=jnp.float32)


def kernel(hidden_states, c_attn_w, c_attn_b, c_proj_w, c_proj_b):
    B, S, E = hidden_states.shape
    n_prog = _NUM_HEADS // _HPP
    x = hidden_states.reshape(S, E)
    cb = c_attn_b.reshape(3 * n_prog, 1, _W)
    pb = c_proj_b.reshape(1, E)
    out = pl.pallas_call(
        _attn_kernel,
        grid=(n_prog,),
        in_specs=[
            pl.BlockSpec((S, E), lambda g: (0, 0)),
            pl.BlockSpec((E, _W), lambda g: (0, g)),
            pl.BlockSpec((E, _W), lambda g: (0, n_prog + g)),
            pl.BlockSpec((E, _W), lambda g: (0, 2 * n_prog + g)),
            pl.BlockSpec((1, 1, _W), lambda g: (g, 0, 0)),
            pl.BlockSpec((1, 1, _W), lambda g: (n_prog + g, 0, 0)),
            pl.BlockSpec((1, 1, _W), lambda g: (2 * n_prog + g, 0, 0)),
            pl.BlockSpec((_W, E), lambda g: (g, 0)),
            pl.BlockSpec((1, E), lambda g: (0, 0)),
        ],
        out_specs=pl.BlockSpec((S, E), lambda g: (0, 0)),
        out_shape=jax.ShapeDtypeStruct((S, E), jnp.float32),
        compiler_params=pltpu.CompilerParams(
            dimension_semantics=("arbitrary",),
            vmem_limit_bytes=100 * 1024 * 1024,
        ),
    )(x, c_attn_w, c_attn_w, c_attn_w, cb, cb, cb, c_proj_w, pb)
    return out.reshape(B, S, E)


# speedup vs baseline: 1.2751x; 1.2751x over previous
import jax
import jax.numpy as jnp
from jax.experimental import pallas as pl
from jax.experimental.pallas import tpu as pltpu

_NUM_HEADS = 12
_HEAD_DIM = 64
_EMBED = 768
_GROUP = 4
_CHUNK = 1024
_HPP = 4
_PAIR = 128
_W = _HPP * _HEAD_DIM
_INV_SQRT = 1.0 / (_HEAD_DIM ** 0.5)
_SCALE = _INV_SQRT * 1.4426950408889634

_DN_T = (((1,), (1,)), ((), ()))
_DN_N = (((1,), (0,)), ((), ()))


def _dequant_keys(k):
    a = jnp.abs(k)
    n = k.shape[1]
    w = a
    for r in (1, 2, 3):
        w = jnp.maximum(w, pltpu.roll(a, n - r, 1))
    lane = jax.lax.broadcasted_iota(jnp.int32, k.shape, 1)
    first = (lane % _GROUP) == 0
    m0 = jnp.where(first, w, 0.0)
    gm = m0
    for r in (1, 2, 3):
        gm = gm + pltpu.roll(m0, r, 1)
    scale = jnp.where(gm > 0.0, gm, 1.0)
    kq = jnp.clip(jnp.round(k / scale), -2.0, 1.0)
    return kq * scale


def _attn_kernel(x_ref, wq_ref, wk_ref, wv_ref, bq_ref, bk_ref, bv_ref,
                 wp_ref, bp_ref, out_ref):
    g = pl.program_id(0)

    @pl.when(g == 0)
    def _init():
        out_ref[...] = jnp.broadcast_to(bp_ref[...], out_ref.shape)

    x = x_ref[...]
    wq, wk, wv = wq_ref[...], wk_ref[...], wv_ref[...]
    bq, bk, bv = bq_ref[0], bk_ref[0], bv_ref[0]
    xb = x.astype(jnp.bfloat16)
    k2 = jax.lax.dot_general(x, wk, _DN_N) + bk
    v2 = jax.lax.dot_general(xb, wv.astype(jnp.bfloat16), _DN_N,
                             preferred_element_type=jnp.float32) + bv
    kd2 = _dequant_keys(k2)
    kd2b = kd2.astype(jnp.bfloat16)
    wqb = wq.astype(jnp.bfloat16)
    wpb = wp_ref[...].astype(jnp.bfloat16)
    vaug = []
    for hh in range(_HPP):
        vp = v2[:, (hh // 2) * _PAIR:(hh // 2) * _PAIR + _PAIR]
        lane = jax.lax.broadcasted_iota(jnp.int32, vp.shape, 1)
        if hh % 2 == 0:
            va = jnp.where(lane < _HEAD_DIM, vp,
                           jnp.where(lane == _HEAD_DIM, 1.0, 0.0))
        else:
            va = jnp.where(lane >= _HEAD_DIM, vp,
                           jnp.where(lane == 0, 1.0, 0.0))
        vaug.append(va.astype(jnp.bfloat16))
    cm = jnp.max(jnp.abs(kd2), axis=0, keepdims=True)
    cmt = jnp.broadcast_to(cm, (_W, _W)).T
    row_i = jax.lax.broadcasted_iota(jnp.int32, (_W, _W), 0)
    col_i = jax.lax.broadcasted_iota(jnp.int32, (_W, _W), 1)
    sel = col_i == (row_i // _HEAD_DIM) * _HEAD_DIM
    cmm = jnp.where(sel, cmt, 0.0)
    seq = x.shape[0]
    for c in range(seq // _CHUNK):
        rows = pl.ds(c * _CHUNK, _CHUNK)
        xcb = xb[c * _CHUNK:(c + 1) * _CHUNK, :]
        q2 = (jax.lax.dot_general(xcb, wqb, _DN_N,
                                  preferred_element_type=jnp.float32)
              + bq) * _SCALE
        q2b = q2.astype(jnp.bfloat16)
        mm = jax.lax.dot_general(jnp.abs(q2), cmm, _DN_N)
        outs = []
        for hh in range(_HPP):
            lanes = slice(hh * _HEAD_DIM, (hh + 1) * _HEAD_DIM)
            s = jax.lax.dot_general(q2b[:, lanes], kd2b[:, lanes], _DN_T,
                                    preferred_element_type=jnp.float32)
            m = mm[:, hh * _HEAD_DIM:hh * _HEAD_DIM + 1]
            e = jnp.exp2(s - m)
            eb = e.astype(jnp.bfloat16)
            o_raw = jax.lax.dot_general(
                eb, vaug[hh], (((1,), (0,)), ((), ())),
                preferred_element_type=jnp.float32)
            if hh % 2 == 0:
                den = o_raw[:, _HEAD_DIM:_HEAD_DIM + 1]
                outs.append(o_raw[:, 0:_HEAD_DIM] / den)
            else:
                den = o_raw[:, 0:1]
                outs.append(o_raw[:, _HEAD_DIM:_PAIR] / den)
        o2 = jnp.concatenate(outs, axis=1).astype(jnp.bfloat16)
        out_ref[rows, :] += jax.lax.dot_general(
            o2, wpb, (((1,), (0,)), ((), ())),
            preferred_element_type=jnp.float32)


def kernel(hidden_states, c_attn_w, c_attn_b, c_proj_w, c_proj_b):
    B, S, E = hidden_states.shape
    n_prog = _NUM_HEADS // _HPP
    x = hidden_states.reshape(S, E)
    cb = c_attn_b.reshape(3 * n_prog, 1, _W)
    pb = c_proj_b.reshape(1, E)
    out = pl.pallas_call(
        _attn_kernel,
        grid=(n_prog,),
        in_specs=[
            pl.BlockSpec((S, E), lambda g: (0, 0)),
            pl.BlockSpec((E, _W), lambda g: (0, g)),
            pl.BlockSpec((E, _W), lambda g: (0, n_prog + g)),
            pl.BlockSpec((E, _W), lambda g: (0, 2 * n_prog + g)),
            pl.BlockSpec((1, 1, _W), lambda g: (g, 0, 0)),
            pl.BlockSpec((1, 1, _W), lambda g: (n_prog + g, 0, 0)),
            pl.BlockSpec((1, 1, _W), lambda g: (2 * n_prog + g, 0, 0)),
            pl.BlockSpec((_W, E), lambda g: (g, 0)),
            pl.BlockSpec((1, E), lambda g: (0, 0)),
        ],
        out_specs=pl.BlockSpec((S, E), lambda g: (0, 0)),
        out_shape=jax.ShapeDtypeStruct((S, E), jnp.float32),
        compiler_params=pltpu.CompilerParams(
            dimension_semantics=("arbitrary",),
            vmem_limit_bytes=100 * 1024 * 1024,
        ),
    )(x, c_attn_w, c_attn_w, c_attn_w, cb, cb, cb, c_proj_w, pb)
    return out.reshape(B, S, E)
